# baseline (speedup 1.0000x reference)
import jax
import jax.numpy as jnp
from jax.experimental import pallas as pl
from jax.experimental.pallas import tpu as pltpu

_B = 2048
_D = 512
_N = 4 * _B
_T = 512
_NT = _N // _T
_KT = _NT // 2 + 1
_TAU = 0.5
_EPS = 1e-12
_LOG2E = 1.4426950408889634
_SCALE = (_LOG2E / _TAU) ** 0.5


def _vred(x):
    return jnp.sum(x.reshape(_T // 16, 16, _T), axis=0,
                   dtype=jnp.bfloat16).astype(jnp.float32)


def _vred_diag(e, eye):
    parts = []
    for c in range(_T // 128):
        slab = e[c * 128:(c + 1) * 128, c * 128:(c + 1) * 128] * eye
        parts.append(jnp.sum(slab.reshape(8, 16, 128), axis=0,
                             dtype=jnp.bfloat16))
    return ((parts[0] + parts[1]) + (parts[2] + parts[3])).astype(jnp.float32)


def _contract(a, b):
    return jax.lax.dot_general(
        a, b, (((1,), (1,)), ((), ())), preferred_element_type=jnp.float32)


def _sim_body(xi_ref, xj_ref, o_ref, rep_ref, eye_ref,
              g0_ref, g1_ref, g2_ref, g3_ref):
    k = pl.program_id(0)

    @pl.when(k == 0)
    def _():
        for t in range(_NT):
            src = xi_ref if t < _NT // 2 else xj_ref
            x = src[(t % (_NT // 2)) * _T:(t % (_NT // 2) + 1) * _T, :]
            n = jnp.sqrt(jnp.sum(x * x, axis=1, keepdims=True))
            rep_ref[t * _T:(t + 1) * _T, :] = (
                x * (_SCALE / jnp.maximum(n, _EPS))).astype(rep_ref.dtype)
        ii = jax.lax.broadcasted_iota(jnp.int32, (128, 128), 0)
        jj = jax.lax.broadcasted_iota(jnp.int32, (128, 128), 1)
        eye_ref[...] = jnp.where(ii == jj, 1.0, 0.0).astype(eye_ref.dtype)
        g0_ref[...] = jnp.zeros_like(g0_ref)
        g1_ref[...] = jnp.zeros_like(g1_ref)
        g2_ref[...] = jnp.zeros_like(g2_ref)
        g3_ref[...] = jnp.zeros_like(g3_ref)

    eye = eye_ref[...]
    s_tot = jnp.zeros((16, _T), jnp.float32)
    d_tot = jnp.zeros((16, 128), jnp.float32)
    for j in range(_NT):
        a = rep_ref[pl.ds(j * _T, _T), :]
        b = rep_ref[pl.ds(((j + k) % _NT) * _T, _T), :]
        e = jnp.exp2(_contract(a, b).astype(jnp.bfloat16))
        s_tot += _vred(e)
        d_tot += _vred_diag(e, eye)

    zs = jnp.zeros((16, _T), jnp.float32)
    zd = jnp.zeros((16, 128), jnp.float32)
    g0_ref[...] += s_tot
    g1_ref[...] += jnp.where((k == 0) | (k == _KT - 1), s_tot, zs)
    g2_ref[...] += jnp.where(k == 0, d_tot, zd)
    g3_ref[...] += jnp.where(
        k == _KT // 2, d_tot + d_tot, jnp.where(k == _KT - 1, d_tot, zd))

    @pl.when(k == _KT - 1)
    def _():
        total = 2.0 * jnp.sum(g0_ref[...]) - jnp.sum(g1_ref[...])
        nominator = jnp.sum(g3_ref[...])
        denominator = total - jnp.sum(g2_ref[...]) - nominator
        loss = -jnp.log(nominator / denominator) * (1.0 / _N)
        o_ref[...] = jnp.full((1, 128), loss, jnp.float32)


def kernel(emb_i, emb_j):
    out = pl.pallas_call(
        _sim_body,
        grid=(_KT,),
        in_specs=[
            pl.BlockSpec((_N // 2, _D), lambda k: (0, 0)),
            pl.BlockSpec((_N // 2, _D), lambda k: (0, 0)),
        ],
        out_specs=pl.BlockSpec((1, 128), lambda k: (0, 0)),
        out_shape=jax.ShapeDtypeStruct((1, 128), jnp.float32),
        scratch_shapes=[
            pltpu.VMEM((_N, _D), jnp.float8_e4m3fn),
            pltpu.VMEM((128, 128), jnp.bfloat16),
            pltpu.VMEM((16, _T), jnp.float32),
            pltpu.VMEM((16, _T), jnp.float32),
            pltpu.VMEM((16, 128), jnp.float32),
            pltpu.VMEM((16, 128), jnp.float32),
        ],
        compiler_params=pltpu.CompilerParams(
            dimension_semantics=("arbitrary",),
            vmem_limit_bytes=56 * 1024 * 1024),
        name="ntxent_sim_reduce",
    )(emb_i, emb_j)
    return out[0, 0]


# speedup vs baseline: 1.0617x; 1.0617x over previous
import jax
import jax.numpy as jnp
from jax.experimental import pallas as pl
from jax.experimental.pallas import tpu as pltpu

_B = 2048
_D = 512
_N = 4 * _B
_T = 512
_NT = _N // _T
_KT = _NT // 2 + 1
_TAU = 0.5
_EPS = 1e-12
_LOG2E = 1.4426950408889634
_SCALE = (_LOG2E / _TAU) ** 0.5


def _red(x):
    r = jnp.sum(x, axis=0, keepdims=True, dtype=x.dtype)
    return (r[:, 0:128] + r[:, 128:256]) + (r[:, 256:384] + r[:, 384:512])


def _red_diag(e, eye):
    parts = []
    for c in range(_T // 128):
        slab = e[c * 128:(c + 1) * 128, c * 128:(c + 1) * 128] * eye
        parts.append(jnp.sum(slab, axis=0, keepdims=True, dtype=e.dtype))
    return (parts[0] + parts[1]) + (parts[2] + parts[3])


def _contract(a, b):
    return jax.lax.dot_general(
        a, b, (((1,), (1,)), ((), ())), preferred_element_type=jnp.float32)


def _exp_tile(rep_ref, j, off):
    a = rep_ref[pl.ds(j * _T, _T), :]
    b = rep_ref[pl.ds(((j + off) % _NT) * _T, _T), :]
    return jnp.exp2(_contract(a, b).astype(jnp.bfloat16))


def _sim_body(xi_ref, xj_ref, o_ref, rep_ref, eye_ref, g_ref):
    k = pl.program_id(0)

    @pl.when(k == 0)
    def _():
        for t in range(_NT):
            src = xi_ref if t < _NT // 2 else xj_ref
            x = src[(t % (_NT // 2)) * _T:(t % (_NT // 2) + 1) * _T, :]
            n = jnp.sqrt(jnp.sum(x * x, axis=1, keepdims=True))
            rep_ref[t * _T:(t + 1) * _T, :] = (
                x * (_SCALE / jnp.maximum(n, _EPS))).astype(rep_ref.dtype)
        ii = jax.lax.broadcasted_iota(jnp.int32, (128, 128), 0)
        jj = jax.lax.broadcasted_iota(jnp.int32, (128, 128), 1)
        eye_ref[...] = jnp.where(ii == jj, 1.0, 0.0).astype(eye_ref.dtype)
        g_ref[...] = jnp.zeros_like(g_ref)

    zero = jnp.zeros((1, 128), jnp.float32)

    @pl.when(k <= 1)
    def _():
        off = 4 * k
        eye = eye_ref[...]
        s_tot = jnp.zeros((1, 128), jnp.float32)
        d_tot = jnp.zeros((1, 128), jnp.float32)
        for j in range(_NT):
            e = _exp_tile(rep_ref, j, off)
            s_tot += _red(e).astype(jnp.float32)
            d_tot += _red_diag(e, eye).astype(jnp.float32)
        g_ref[0:1, :] += s_tot
        g_ref[1:2, :] += jnp.where(k == 0, s_tot, zero)
        g_ref[2:3, :] += jnp.where(k == 0, d_tot, zero)
        g_ref[3:4, :] += jnp.where(k == 1, d_tot + d_tot, zero)

    @pl.when(k == 2)
    def _():
        eye = eye_ref[...]
        s_tot = jnp.zeros((1, 128), jnp.float32)
        d_tot = jnp.zeros((1, 128), jnp.float32)
        for j in range(_NT // 2):
            e = _exp_tile(rep_ref, j, 8)
            s_tot += _red(e).astype(jnp.float32)
            d_tot += _red_diag(e, eye).astype(jnp.float32)
        g_ref[0:1, :] += s_tot
        g_ref[3:4, :] += d_tot + d_tot

    @pl.when(k >= 3)
    def _():
        off = jnp.where(k < 6, k - 2, k - 1)
        s_tot = jnp.zeros((1, 128), jnp.float32)
        for j in range(_NT):
            e = _exp_tile(rep_ref, j, off)
            s_tot += _red(e).astype(jnp.float32)
        g_ref[0:1, :] += s_tot

    @pl.when(k == _KT - 1)
    def _():
        g = g_ref[...]
        t = jnp.sum(g, axis=1, keepdims=True)
        total = 2.0 * t[0, 0] - t[1, 0]
        nominator = t[3, 0]
        denominator = total - t[2, 0] - nominator
        loss = -jnp.log(nominator / denominator) * (1.0 / _N)
        o_ref[...] = jnp.full((1, 128), loss, jnp.float32)


def kernel(emb_i, emb_j):
    out = pl.pallas_call(
        _sim_body,
        grid=(_KT,),
        in_specs=[
            pl.BlockSpec((_N // 2, _D), lambda k: (0, 0)),
            pl.BlockSpec((_N // 2, _D), lambda k: (0, 0)),
        ],
        out_specs=pl.BlockSpec((1, 128), lambda k: (0, 0)),
        out_shape=jax.ShapeDtypeStruct((1, 128), jnp.float32),
        scratch_shapes=[
            pltpu.VMEM((_N, _D), jnp.float8_e4m3fn),
            pltpu.VMEM((128, 128), jnp.bfloat16),
            pltpu.VMEM((4, 128), jnp.float32),
        ],
        compiler_params=pltpu.CompilerParams(
            dimension_semantics=("arbitrary",),
            vmem_limit_bytes=56 * 1024 * 1024),
        name="ntxent_sim_reduce",
    )(emb_i, emb_j)
    return out[0, 0]
